# Initial kernel scaffold; baseline (speedup 1.0000x reference)
import jax
import jax.numpy as jnp
from jax.experimental import pallas as pl


def kernel(x, edge_index, edge_type, basis1, comp1, root1, bias1, basis2, comp2, root2, bias2):
    raise NotImplementedError("write your pallas kernel here")


# speedup vs baseline: 3.7674x; 3.7674x over previous
import functools

import jax
import jax.numpy as jnp
from jax import lax
from jax.experimental import pallas as pl
from jax.experimental.pallas import tpu as pltpu
from jax.experimental.pallas import tpu_sc as plsc

N = 10000
REL = 40
EMB = 128
HID = 64
E = 160000
CHUNK = 5000
NSEG = N * REL
NC, NS = 2, 16
NW = NC * NS
E_PAD = 163840
CPT = E_PAD // NS
EPW = E_PAD // NW
CSTRIPE = NSEG // NS
ABIG = 640
ALAST = N - 15 * ABIG
F32 = jnp.float32
I32 = jnp.int32

_mesh = plsc.VectorSubcoreMesh(core_axis_name="c", subcore_axis_name="s")


def _z16():
    return jnp.zeros((16,), F32)


def _fill_zero_1d(ref, n):
    def body(i, _):
        ref[pl.ds(i * 16, 16)] = _z16()
        return 0
    lax.fori_loop(0, n // 16, body, 0)


def _fill_zero_2d(ref, rows, cols):
    def body(i, _):
        r = i // (cols // 16)
        q = i % (cols // 16)
        ref[r, pl.ds(q * 16, 16)] = _z16()
        return 0
    lax.fori_loop(0, rows * (cols // 16), body, 0)


def _zero_stripe_2d(zrows, tab, sid):
    @pl.when(sid < 15)
    def _():
        pltpu.sync_copy(zrows, tab.at[pl.ds(sid * ABIG, 256)])
        pltpu.sync_copy(zrows, tab.at[pl.ds(sid * ABIG + 256, 256)])
        pltpu.sync_copy(zrows.at[pl.ds(0, 128)],
                        tab.at[pl.ds(sid * ABIG + 512, 128)])

    @pl.when(sid == 15)
    def _():
        pltpu.sync_copy(zrows, tab.at[pl.ds(15 * ABIG, 256)])
        pltpu.sync_copy(zrows.at[pl.ds(0, 144)],
                        tab.at[pl.ds(15 * ABIG + 256, 144)])


def _copy_stripe_out(tab, out, sid):
    @pl.when(sid < 15)
    def _():
        pltpu.sync_copy(tab.at[pl.ds(sid * ABIG, ABIG)],
                        out.at[pl.ds(sid * ABIG, ABIG)])

    @pl.when(sid == 15)
    def _():
        pltpu.sync_copy(tab.at[pl.ds(15 * ABIG, ALAST)],
                        out.at[pl.ds(15 * ABIG, ALAST)])


@functools.partial(
    pl.kernel,
    out_type=(
        jax.ShapeDtypeStruct((E_PAD,), F32),
        jax.ShapeDtypeStruct((E_PAD,), F32),
        jax.ShapeDtypeStruct((NW * 32,), F32),
    ),
    mesh=_mesh,
    scratch_types=(
        pltpu.VMEM_SHARED((NSEG,), F32),
        pltpu.VMEM((CPT,), I32),
        pltpu.VMEM((CPT,), I32),
        pltpu.VMEM((CPT,), I32),
        pltpu.VMEM((CPT,), F32),
        pltpu.VMEM((8000,), F32),
        pltpu.VMEM((2048,), I32),
        pltpu.VMEM((2048,), F32),
        pltpu.VMEM((2048,), F32),
        pltpu.VMEM((2048,), F32),
        pltpu.VMEM((32,), F32),
    ),
)
def _sc_counts(src_hbm, dst_hbm, typ_hbm,
               af_hbm, ac_hbm, touch_hbm,
               ctab, srcb, dstb, typb, wantall, zc,
               segc, valc, cfb, alb, touchb):
    cid = lax.axis_index("c")
    sid = lax.axis_index("s")
    iota16 = lax.iota(I32, 16)

    _fill_zero_1d(zc, 8000)
    for j in range(3):
        pltpu.sync_copy(zc.at[pl.ds(0, 8000)],
                        ctab.at[pl.ds(sid * CSTRIPE + j * 8000, 8000)])
    pltpu.sync_copy(zc.at[pl.ds(0, 1000)],
                    ctab.at[pl.ds(sid * CSTRIPE + 24000, 1000)])
    ebase = sid * CPT
    pltpu.sync_copy(src_hbm.at[pl.ds(ebase, CPT)], srcb)
    pltpu.sync_copy(dst_hbm.at[pl.ds(ebase, CPT)], dstb)
    pltpu.sync_copy(typ_hbm.at[pl.ds(ebase, CPT)], typb)
    plsc.subcore_barrier()

    def count_chunk(ch, touch):
        t0a, t1a = touch
        base = ch * 2048

        def body(i, carry):
            a0, a1 = carry
            off = base + i * 16
            s = srcb[pl.ds(off, 16)]
            d = dstb[pl.ds(off, 16)]
            t = typb[pl.ds(off, 16)]
            segc[pl.ds(i * 16, 16)] = d * REL + t
            valid = (ebase + off + iota16) < E
            crossb = (s >= CHUNK) != (d >= CHUNK)
            want = jnp.where(cid == 0, valid, valid & crossb)
            wf = jnp.where(want, 1.0, 0.0)
            valc[pl.ds(i * 16, 16)] = wf
            wantall[pl.ds(off, 16)] = wf
            a0 = a0 + jnp.where(((s < CHUNK) | (d < CHUNK)) & valid, 1.0, 0.0)
            a1 = a1 + jnp.where(((s >= CHUNK) | (d >= CHUNK)) & valid, 1.0, 0.0)
            return a0, a1

        t0a, t1a = lax.fori_loop(0, 128, body, (t0a, t1a))
        pltpu.sync_copy(valc, ctab.at[segc], add=True)
        return t0a, t1a

    t0a, t1a = lax.fori_loop(0, CPT // 2048, count_chunk, (_z16(), _z16()))
    zero_on_sc1 = jnp.where(cid == 0, 1.0, 0.0)
    touchb[pl.ds(0, 16)] = zero_on_sc1 * t0a
    touchb[pl.ds(16, 16)] = zero_on_sc1 * t1a
    pltpu.sync_copy(touchb, touch_hbm.at[pl.ds((sid * NC + cid) * 32, 32)])
    plsc.subcore_barrier()

    def alpha_chunk(ch, _):
        base = ch * 2048

        def mkseg(i, _c):
            off = base + i * 16
            d = dstb[pl.ds(off, 16)]
            t = typb[pl.ds(off, 16)]
            segc[pl.ds(i * 16, 16)] = d * REL + t
            return 0

        lax.fori_loop(0, 128, mkseg, 0)
        pltpu.sync_copy(ctab.at[segc], cfb)

        def mkalpha(i, _c):
            c = cfb[pl.ds(i * 16, 16)]
            w = wantall[pl.ds(base + i * 16, 16)]
            alb[pl.ds(i * 16, 16)] = w / jnp.maximum(c, 1.0)
            return 0

        lax.fori_loop(0, 128, mkalpha, 0)

        @pl.when(cid == 0)
        def _():
            pltpu.sync_copy(alb, af_hbm.at[pl.ds(ebase + base, 2048)])

        @pl.when(cid == 1)
        def _():
            pltpu.sync_copy(alb, ac_hbm.at[pl.ds(ebase + base, 2048)])
        return 0

    lax.fori_loop(0, CPT // 2048, alpha_chunk, 0)


@functools.partial(
    pl.kernel,
    out_type=jax.ShapeDtypeStruct((NC, N, 2 * HID), F32),
    mesh=_mesh,
    scratch_types=(
        pltpu.VMEM_SHARED((N, 2 * HID), F32),
        pltpu.VMEM((256,), I32),
        pltpu.VMEM((256,), I32),
        pltpu.VMEM((256,), I32),
        pltpu.VMEM((256,), F32),
        pltpu.VMEM((256,), F32),
        pltpu.VMEM((256,), I32),
        pltpu.VMEM((256,), I32),
        pltpu.VMEM((256, 2 * HID), F32),
    ),
)
def _sc_conv1(src_hbm, dst_hbm, typ_hbm, y1_hbm, af_hbm, ac_hbm,
              apart_hbm,
              atab, srcb, dstb, typb, afb, acb, gidx, halfb, rows):
    cid = lax.axis_index("c")
    sid = lax.axis_index("s")

    _fill_zero_2d(rows, 256, 2 * HID)
    _zero_stripe_2d(rows, atab, sid)
    ebase = (cid * NS + sid) * EPW
    plsc.subcore_barrier()

    def msg_chunk(ch, _):
        base = ebase + ch * 256
        pltpu.sync_copy(src_hbm.at[pl.ds(base, 256)], srcb)
        pltpu.sync_copy(dst_hbm.at[pl.ds(base, 256)], dstb)
        pltpu.sync_copy(typ_hbm.at[pl.ds(base, 256)], typb)
        pltpu.sync_copy(af_hbm.at[pl.ds(base, 256)], afb)
        pltpu.sync_copy(ac_hbm.at[pl.ds(base, 256)], acb)

        def prep(i, _c):
            s = srcb[pl.ds(i * 16, 16)]
            t = typb[pl.ds(i * 16, 16)]
            g = s * REL + t
            gidx[pl.ds(i * 16, 16)] = lax.shift_right_logical(g, 1)
            halfb[pl.ds(i * 16, 16)] = (g & 1) * HID
            return 0

        lax.fori_loop(0, 16, prep, 0)
        pltpu.sync_copy(y1_hbm.at[gidx], rows)

        def scale(grp, _c):
            av = afb[pl.ds(grp * 16, 16)]
            bv = acb[pl.ds(grp * 16, 16)]
            hv = halfb[pl.ds(grp * 16, 16)]
            for l in range(16):
                e = grp * 16 + l
                a = av[l]
                b = bv[l]
                h = hv[l]
                for q in range(HID // 16):
                    v = rows[e, pl.ds(h + q * 16, 16)]
                    rows[e, pl.ds(q * 16, 16)] = v * a
                    rows[e, pl.ds(HID + q * 16, 16)] = v * b
            return 0

        lax.fori_loop(0, 16, scale, 0)
        pltpu.sync_copy(rows, atab.at[dstb], add=True)
        return 0

    lax.fori_loop(0, EPW // 256, msg_chunk, 0)
    plsc.subcore_barrier()
    _copy_stripe_out(atab, apart_hbm.at[cid], sid)


@functools.partial(
    pl.kernel,
    out_type=jax.ShapeDtypeStruct((NC, N, EMB), F32),
    mesh=_mesh,
    scratch_types=(
        pltpu.VMEM_SHARED((N, EMB), F32),
        pltpu.VMEM((256,), I32),
        pltpu.VMEM((256,), I32),
        pltpu.VMEM((256,), I32),
        pltpu.VMEM((256,), F32),
        pltpu.VMEM((256,), I32),
        pltpu.VMEM((256, EMB), F32),
    ),
)
def _sc_conv2(src_hbm, dst_hbm, typ_hbm, y2_hbm, af_hbm,
              a2_hbm,
              a2tab, srcb, dstb, typb, afb, gidx, rows):
    cid = lax.axis_index("c")
    sid = lax.axis_index("s")

    _fill_zero_2d(rows, 256, EMB)
    _zero_stripe_2d(rows, a2tab, sid)
    ebase = (cid * NS + sid) * EPW
    plsc.subcore_barrier()

    def msg_chunk(ch, _):
        base = ebase + ch * 256
        pltpu.sync_copy(src_hbm.at[pl.ds(base, 256)], srcb)
        pltpu.sync_copy(dst_hbm.at[pl.ds(base, 256)], dstb)
        pltpu.sync_copy(typ_hbm.at[pl.ds(base, 256)], typb)
        pltpu.sync_copy(af_hbm.at[pl.ds(base, 256)], afb)

        def prep(i, _c):
            s = srcb[pl.ds(i * 16, 16)]
            d = dstb[pl.ds(i * 16, 16)]
            t = typb[pl.ds(i * 16, 16)]
            crossb = (s >= CHUNK) != (d >= CHUNK)
            s2 = jnp.where(crossb, s + N, s)
            gidx[pl.ds(i * 16, 16)] = s2 * REL + t
            return 0

        lax.fori_loop(0, 16, prep, 0)
        pltpu.sync_copy(y2_hbm.at[gidx], rows)

        def scale(grp, _c):
            av = afb[pl.ds(grp * 16, 16)]
            for l in range(16):
                e = grp * 16 + l
                a = av[l]
                for q in range(EMB // 16):
                    rows[e, pl.ds(q * 16, 16)] = rows[e, pl.ds(q * 16, 16)] * a
            return 0

        lax.fori_loop(0, 16, scale, 0)
        pltpu.sync_copy(rows, a2tab.at[dstb], add=True)
        return 0

    lax.fori_loop(0, EPW // 256, msg_chunk, 0)
    plsc.subcore_barrier()
    _copy_stripe_out(a2tab, a2_hbm.at[cid], sid)


def _tc_prep_body(x_ref, w1_ref, r1_ref, y1_ref, xr1_ref):
    xb = x_ref[...]
    y1_ref[...] = jnp.dot(xb, w1_ref[...], preferred_element_type=F32)
    xr1_ref[...] = jnp.dot(xb, r1_ref[...], preferred_element_type=F32)


def _tc_prep(x, w1cat, root1):
    rb = 1000
    return pl.pallas_call(
        _tc_prep_body,
        grid=(N // rb,),
        in_specs=[
            pl.BlockSpec((rb, EMB), lambda i: (i, 0)),
            pl.BlockSpec((EMB, REL * HID), lambda i: (0, 0)),
            pl.BlockSpec((EMB, HID), lambda i: (0, 0)),
        ],
        out_specs=[
            pl.BlockSpec((rb, REL * HID), lambda i: (i, 0)),
            pl.BlockSpec((rb, HID), lambda i: (i, 0)),
        ],
        out_shape=[
            jax.ShapeDtypeStruct((N, REL * HID), F32),
            jax.ShapeDtypeStruct((N, HID), F32),
        ],
    )(x, w1cat, root1)


def _tc_mid_body(a_ref, xr1_ref, b1_ref, w2_ref, r2_ref, b2_ref,
                 y2_ref, h0_ref):
    k = pl.program_id(0)
    asum = a_ref[0] + a_ref[1]
    agg = jnp.where(k == 0, asum[:, :HID], asum[:, HID:])
    t = jnp.maximum(agg + xr1_ref[...] + b1_ref[...], 0.0)
    y2_ref[0] = jnp.dot(t, w2_ref[...], preferred_element_type=F32)
    h0_ref[0] = jnp.dot(t, r2_ref[...], preferred_element_type=F32) + b2_ref[...]


def _tc_mid(apart, xr1, bias1, w2cat, root2, bias2):
    rb = 400
    return pl.pallas_call(
        _tc_mid_body,
        grid=(2, N // rb),
        in_specs=[
            pl.BlockSpec((2, rb, 2 * HID), lambda k, i: (0, i, 0)),
            pl.BlockSpec((rb, HID), lambda k, i: (i, 0)),
            pl.BlockSpec((1, HID), lambda k, i: (0, 0)),
            pl.BlockSpec((HID, REL * EMB), lambda k, i: (0, 0)),
            pl.BlockSpec((HID, EMB), lambda k, i: (0, 0)),
            pl.BlockSpec((1, EMB), lambda k, i: (0, 0)),
        ],
        out_specs=[
            pl.BlockSpec((1, rb, REL * EMB), lambda k, i: (k, i, 0)),
            pl.BlockSpec((1, rb, EMB), lambda k, i: (k, i, 0)),
        ],
        out_shape=[
            jax.ShapeDtypeStruct((2, N, REL * EMB), F32),
            jax.ShapeDtypeStruct((2, N, EMB), F32),
        ],
    )(apart, xr1, bias1, w2cat, root2, bias2)


def _tc_final_body(a2_ref, h0_ref, x_ref, touch_ref, h_ref):
    tch = touch_ref[...]
    t0 = jnp.sum(tch[:, :16]) > 0.0
    t1 = jnp.sum(tch[:, 16:]) > 0.0
    i = pl.program_id(0)
    tk = jnp.where(i < CHUNK // 1000, t0, t1)
    val = a2_ref[0] + a2_ref[1] + h0_ref[...]
    h_ref[...] = jnp.where(tk, val, x_ref[...])


def _tc_final(a2p, h0, x, touch):
    rb = 1000
    return pl.pallas_call(
        _tc_final_body,
        grid=(N // rb,),
        in_specs=[
            pl.BlockSpec((2, rb, EMB), lambda i: (0, i, 0)),
            pl.BlockSpec((rb, EMB), lambda i: (i, 0)),
            pl.BlockSpec((rb, EMB), lambda i: (i, 0)),
            pl.BlockSpec((NW, 32), lambda i: (0, 0)),
        ],
        out_specs=pl.BlockSpec((rb, EMB), lambda i: (i, 0)),
        out_shape=jax.ShapeDtypeStruct((N, EMB), F32),
    )(a2p, h0, x, touch)


def kernel(x, edge_index, edge_type, basis1, comp1, root1, bias1,
           basis2, comp2, root2, bias2):
    src = edge_index[0].astype(I32)
    dst = edge_index[1].astype(I32)
    typ = edge_type.astype(I32)
    pad = E_PAD - E
    src = jnp.pad(src, (0, pad))
    dst = jnp.pad(dst, (0, pad))
    typ = jnp.pad(typ, (0, pad))

    w1cat = jnp.einsum('rb,bio->iro', comp1, basis1).reshape(EMB, REL * HID)
    w2cat = jnp.einsum('rb,bio->iro', comp2, basis2).reshape(HID, REL * EMB)

    y1, xr1 = _tc_prep(x, w1cat, root1)
    alphaf, alphac, touch = _sc_counts(src, dst, typ)
    apart = _sc_conv1(src, dst, typ, y1.reshape(N * REL // 2, 2 * HID),
                      alphaf, alphac)
    y2, h0 = _tc_mid(apart, xr1, bias1.reshape(1, HID), w2cat, root2,
                     bias2.reshape(1, EMB))
    a2p = _sc_conv2(src, dst, typ, y2.reshape(2 * N * REL, EMB), alphaf)
    return _tc_final(a2p, h0[0], x, touch.reshape(NW, 32))


# speedup vs baseline: 4.9063x; 1.3023x over previous
import functools

import jax
import jax.numpy as jnp
from jax import lax
from jax.experimental import pallas as pl
from jax.experimental.pallas import tpu as pltpu
from jax.experimental.pallas import tpu_sc as plsc

N = 10000
REL = 40
EMB = 128
HID = 64
E = 160000
CHUNK = 5000
NSEG = N * REL
NC, NS = 2, 16
NW = NC * NS
E_PAD = 163840
CPT = E_PAD // NS
EPW = E_PAD // NW
CSTRIPE = NSEG // NS
ABIG = 640
ALAST = N - 15 * ABIG
F32 = jnp.float32
I32 = jnp.int32

_mesh = plsc.VectorSubcoreMesh(core_axis_name="c", subcore_axis_name="s")


def _z16():
    return jnp.zeros((16,), F32)


def _fill_zero_1d(ref, n):
    def body(i, _):
        ref[pl.ds(i * 16, 16)] = _z16()
        return 0
    lax.fori_loop(0, n // 16, body, 0)


def _fill_zero_2d(ref, rows, cols):
    def body(i, _):
        r = i // (cols // 16)
        q = i % (cols // 16)
        ref[r, pl.ds(q * 16, 16)] = _z16()
        return 0
    lax.fori_loop(0, rows * (cols // 16), body, 0)


def _zero_stripe_2d(zrows, tab, sid):
    @pl.when(sid < 15)
    def _():
        pltpu.sync_copy(zrows, tab.at[pl.ds(sid * ABIG, 256)])
        pltpu.sync_copy(zrows, tab.at[pl.ds(sid * ABIG + 256, 256)])
        pltpu.sync_copy(zrows.at[pl.ds(0, 128)],
                        tab.at[pl.ds(sid * ABIG + 512, 128)])

    @pl.when(sid == 15)
    def _():
        pltpu.sync_copy(zrows, tab.at[pl.ds(15 * ABIG, 256)])
        pltpu.sync_copy(zrows.at[pl.ds(0, 144)],
                        tab.at[pl.ds(15 * ABIG + 256, 144)])


def _copy_stripe_out(tab, out, sid):
    @pl.when(sid < 15)
    def _():
        pltpu.sync_copy(tab.at[pl.ds(sid * ABIG, ABIG)],
                        out.at[pl.ds(sid * ABIG, ABIG)])

    @pl.when(sid == 15)
    def _():
        pltpu.sync_copy(tab.at[pl.ds(15 * ABIG, ALAST)],
                        out.at[pl.ds(15 * ABIG, ALAST)])


@functools.partial(
    pl.kernel,
    out_type=(
        jax.ShapeDtypeStruct((E_PAD,), F32),
        jax.ShapeDtypeStruct((E_PAD,), F32),
        jax.ShapeDtypeStruct((NW * 32,), F32),
    ),
    mesh=_mesh,
    scratch_types=(
        pltpu.VMEM_SHARED((NSEG,), F32),
        pltpu.VMEM((CPT,), I32),
        pltpu.VMEM((CPT,), I32),
        pltpu.VMEM((CPT,), I32),
        pltpu.VMEM((CPT,), F32),
        pltpu.VMEM((8000,), F32),
        pltpu.VMEM((2048,), I32),
        pltpu.VMEM((2048,), F32),
        pltpu.VMEM((2048,), F32),
        pltpu.VMEM((2048,), F32),
        pltpu.VMEM((32,), F32),
    ),
)
def _sc_counts(src_hbm, dst_hbm, typ_hbm,
               af_hbm, ac_hbm, touch_hbm,
               ctab, srcb, dstb, typb, wantall, zc,
               segc, valc, cfb, alb, touchb):
    cid = lax.axis_index("c")
    sid = lax.axis_index("s")
    iota16 = lax.iota(I32, 16)

    _fill_zero_1d(zc, 8000)
    for j in range(3):
        pltpu.sync_copy(zc.at[pl.ds(0, 8000)],
                        ctab.at[pl.ds(sid * CSTRIPE + j * 8000, 8000)])
    pltpu.sync_copy(zc.at[pl.ds(0, 1000)],
                    ctab.at[pl.ds(sid * CSTRIPE + 24000, 1000)])
    ebase = sid * CPT
    pltpu.sync_copy(src_hbm.at[pl.ds(ebase, CPT)], srcb)
    pltpu.sync_copy(dst_hbm.at[pl.ds(ebase, CPT)], dstb)
    pltpu.sync_copy(typ_hbm.at[pl.ds(ebase, CPT)], typb)
    plsc.subcore_barrier()

    def count_chunk(ch, touch):
        t0a, t1a = touch
        base = ch * 2048

        def body(i, carry):
            a0, a1 = carry
            off = base + i * 16
            s = srcb[pl.ds(off, 16)]
            d = dstb[pl.ds(off, 16)]
            t = typb[pl.ds(off, 16)]
            segc[pl.ds(i * 16, 16)] = d * REL + t
            valid = (ebase + off + iota16) < E
            crossb = (s >= CHUNK) != (d >= CHUNK)
            want = jnp.where(cid == 0, valid, valid & crossb)
            wf = jnp.where(want, 1.0, 0.0)
            valc[pl.ds(i * 16, 16)] = wf
            wantall[pl.ds(off, 16)] = wf
            a0 = a0 + jnp.where(((s < CHUNK) | (d < CHUNK)) & valid, 1.0, 0.0)
            a1 = a1 + jnp.where(((s >= CHUNK) | (d >= CHUNK)) & valid, 1.0, 0.0)
            return a0, a1

        t0a, t1a = lax.fori_loop(0, 128, body, (t0a, t1a))
        pltpu.sync_copy(valc, ctab.at[segc], add=True)
        return t0a, t1a

    t0a, t1a = lax.fori_loop(0, CPT // 2048, count_chunk, (_z16(), _z16()))
    zero_on_sc1 = jnp.where(cid == 0, 1.0, 0.0)
    touchb[pl.ds(0, 16)] = zero_on_sc1 * t0a
    touchb[pl.ds(16, 16)] = zero_on_sc1 * t1a
    pltpu.sync_copy(touchb, touch_hbm.at[pl.ds((sid * NC + cid) * 32, 32)])
    plsc.subcore_barrier()

    def alpha_chunk(ch, _):
        base = ch * 2048

        def mkseg(i, _c):
            off = base + i * 16
            d = dstb[pl.ds(off, 16)]
            t = typb[pl.ds(off, 16)]
            segc[pl.ds(i * 16, 16)] = d * REL + t
            return 0

        lax.fori_loop(0, 128, mkseg, 0)
        pltpu.sync_copy(ctab.at[segc], cfb)

        def mkalpha(i, _c):
            c = cfb[pl.ds(i * 16, 16)]
            w = wantall[pl.ds(base + i * 16, 16)]
            alb[pl.ds(i * 16, 16)] = w / jnp.maximum(c, 1.0)
            return 0

        lax.fori_loop(0, 128, mkalpha, 0)

        @pl.when(cid == 0)
        def _():
            pltpu.sync_copy(alb, af_hbm.at[pl.ds(ebase + base, 2048)])

        @pl.when(cid == 1)
        def _():
            pltpu.sync_copy(alb, ac_hbm.at[pl.ds(ebase + base, 2048)])
        return 0

    lax.fori_loop(0, CPT // 2048, alpha_chunk, 0)


@functools.partial(
    pl.kernel,
    out_type=jax.ShapeDtypeStruct((NC, N, 2 * HID), F32),
    mesh=_mesh,
    scratch_types=(
        pltpu.VMEM_SHARED((N, 2 * HID), F32),
        pltpu.VMEM((256,), I32),
        pltpu.VMEM((256,), I32),
        pltpu.VMEM((256,), I32),
        pltpu.VMEM((256,), F32),
        pltpu.VMEM((256,), F32),
        pltpu.VMEM((256,), I32),
        pltpu.VMEM((256,), I32),
        pltpu.VMEM((256, 2 * HID), F32),
    ),
)
def _sc_conv1(src_hbm, dst_hbm, typ_hbm, y1_hbm, af_hbm, ac_hbm,
              apart_hbm,
              atab, srcb, dstb, typb, afb, acb, gidx, halfb, rows):
    cid = lax.axis_index("c")
    sid = lax.axis_index("s")

    _fill_zero_2d(rows, 256, 2 * HID)
    _zero_stripe_2d(rows, atab, sid)
    ebase = (cid * NS + sid) * EPW
    plsc.subcore_barrier()

    def msg_chunk(ch, _):
        base = ebase + ch * 256
        pltpu.sync_copy(src_hbm.at[pl.ds(base, 256)], srcb)
        pltpu.sync_copy(dst_hbm.at[pl.ds(base, 256)], dstb)
        pltpu.sync_copy(typ_hbm.at[pl.ds(base, 256)], typb)
        pltpu.sync_copy(af_hbm.at[pl.ds(base, 256)], afb)
        pltpu.sync_copy(ac_hbm.at[pl.ds(base, 256)], acb)

        def prep(i, _c):
            s = srcb[pl.ds(i * 16, 16)]
            t = typb[pl.ds(i * 16, 16)]
            gidx[pl.ds(i * 16, 16)] = lax.shift_right_logical(t, 1) * N + s
            halfb[pl.ds(i * 16, 16)] = (t & 1) * HID
            return 0

        lax.fori_loop(0, 16, prep, 0)
        pltpu.sync_copy(y1_hbm.at[gidx], rows)

        def scale(grp, _c):
            av = afb[pl.ds(grp * 16, 16)]
            bv = acb[pl.ds(grp * 16, 16)]
            hv = halfb[pl.ds(grp * 16, 16)]
            for l in range(16):
                e = grp * 16 + l
                a = av[l]
                b = bv[l]
                h = hv[l]
                for q in range(HID // 16):
                    v = rows[e, pl.ds(h + q * 16, 16)]
                    rows[e, pl.ds(q * 16, 16)] = v * a
                    rows[e, pl.ds(HID + q * 16, 16)] = v * b
            return 0

        lax.fori_loop(0, 16, scale, 0)
        pltpu.sync_copy(rows, atab.at[dstb], add=True)
        return 0

    lax.fori_loop(0, EPW // 256, msg_chunk, 0)
    plsc.subcore_barrier()
    _copy_stripe_out(atab, apart_hbm.at[cid], sid)


@functools.partial(
    pl.kernel,
    out_type=jax.ShapeDtypeStruct((NC, N, EMB), F32),
    mesh=_mesh,
    scratch_types=(
        pltpu.VMEM_SHARED((N, EMB), F32),
        pltpu.VMEM((256,), I32),
        pltpu.VMEM((256,), I32),
        pltpu.VMEM((256,), I32),
        pltpu.VMEM((256,), F32),
        pltpu.VMEM((256,), I32),
        pltpu.VMEM((256, EMB), F32),
    ),
)
def _sc_conv2(src_hbm, dst_hbm, typ_hbm, y2_hbm, af_hbm,
              a2_hbm,
              a2tab, srcb, dstb, typb, afb, gidx, rows):
    cid = lax.axis_index("c")
    sid = lax.axis_index("s")

    _fill_zero_2d(rows, 256, EMB)
    _zero_stripe_2d(rows, a2tab, sid)
    ebase = (cid * NS + sid) * EPW
    plsc.subcore_barrier()

    def msg_chunk(ch, _):
        base = ebase + ch * 256
        pltpu.sync_copy(src_hbm.at[pl.ds(base, 256)], srcb)
        pltpu.sync_copy(dst_hbm.at[pl.ds(base, 256)], dstb)
        pltpu.sync_copy(typ_hbm.at[pl.ds(base, 256)], typb)
        pltpu.sync_copy(af_hbm.at[pl.ds(base, 256)], afb)

        def prep(i, _c):
            s = srcb[pl.ds(i * 16, 16)]
            d = dstb[pl.ds(i * 16, 16)]
            t = typb[pl.ds(i * 16, 16)]
            crossb = (s >= CHUNK) != (d >= CHUNK)
            s2 = jnp.where(crossb, s + N, s)
            gidx[pl.ds(i * 16, 16)] = t * (2 * N) + s2
            return 0

        lax.fori_loop(0, 16, prep, 0)
        pltpu.sync_copy(y2_hbm.at[gidx], rows)

        def scale(grp, _c):
            av = afb[pl.ds(grp * 16, 16)]
            for l in range(16):
                e = grp * 16 + l
                a = av[l]
                for q in range(EMB // 16):
                    rows[e, pl.ds(q * 16, 16)] = rows[e, pl.ds(q * 16, 16)] * a
            return 0

        lax.fori_loop(0, 16, scale, 0)
        pltpu.sync_copy(rows, a2tab.at[dstb], add=True)
        return 0

    lax.fori_loop(0, EPW // 256, msg_chunk, 0)
    plsc.subcore_barrier()
    _copy_stripe_out(a2tab, a2_hbm.at[cid], sid)


def _tc_prep_body(x_ref, w1_ref, y1_ref):
    y1_ref[0] = jnp.dot(x_ref[...], w1_ref[...], preferred_element_type=F32)


def _tc_prep(x, w1cat):
    rb = 1000
    return pl.pallas_call(
        _tc_prep_body,
        grid=(N // rb, REL // 2),
        in_specs=[
            pl.BlockSpec((rb, EMB), lambda i, p: (i, 0)),
            pl.BlockSpec((EMB, EMB), lambda i, p: (0, p)),
        ],
        out_specs=pl.BlockSpec((1, rb, EMB), lambda i, p: (p, i, 0)),
        out_shape=jax.ShapeDtypeStruct((REL // 2, N, EMB), F32),
    )(x, w1cat)


def _tc_mid_body(a_ref, x_ref, r1_ref, b1_ref, w2_ref, r2_ref, b2_ref,
                 y2_ref, h0_ref):
    k = pl.program_id(0)
    asum = a_ref[0] + a_ref[1]
    agg = jnp.where(k == 0, asum[:, :HID], asum[:, HID:])
    xr1 = jnp.dot(x_ref[...], r1_ref[...], preferred_element_type=F32)
    t = jnp.maximum(agg + xr1 + b1_ref[...], 0.0)
    y2 = jnp.dot(t, w2_ref[...], preferred_element_type=F32)
    for r in range(REL):
        y2_ref[r] = y2[:, r * EMB:(r + 1) * EMB]
    h0_ref[0] = jnp.dot(t, r2_ref[...], preferred_element_type=F32) + b2_ref[...]


def _tc_mid(apart, x, root1, bias1, w2cat, root2, bias2):
    rb = 400
    return pl.pallas_call(
        _tc_mid_body,
        grid=(2, N // rb),
        in_specs=[
            pl.BlockSpec((2, rb, 2 * HID), lambda k, i: (0, i, 0)),
            pl.BlockSpec((rb, EMB), lambda k, i: (i, 0)),
            pl.BlockSpec((EMB, HID), lambda k, i: (0, 0)),
            pl.BlockSpec((1, HID), lambda k, i: (0, 0)),
            pl.BlockSpec((HID, REL * EMB), lambda k, i: (0, 0)),
            pl.BlockSpec((HID, EMB), lambda k, i: (0, 0)),
            pl.BlockSpec((1, EMB), lambda k, i: (0, 0)),
        ],
        out_specs=[
            pl.BlockSpec((REL, rb, EMB),
                         lambda k, i: (0, k * (N // 400) + i, 0)),
            pl.BlockSpec((1, rb, EMB), lambda k, i: (k, i, 0)),
        ],
        out_shape=[
            jax.ShapeDtypeStruct((REL, 2 * N, EMB), F32),
            jax.ShapeDtypeStruct((2, N, EMB), F32),
        ],
    )(apart, x, root1, bias1, w2cat, root2, bias2)


def _tc_final_body(a2_ref, h0_ref, x_ref, touch_ref, h_ref):
    tch = touch_ref[...]
    t0 = jnp.sum(tch[:, :16]) > 0.0
    t1 = jnp.sum(tch[:, 16:]) > 0.0
    i = pl.program_id(0)
    tk = jnp.where(i < CHUNK // 1000, t0, t1)
    val = a2_ref[0] + a2_ref[1] + h0_ref[...]
    h_ref[...] = jnp.where(tk, val, x_ref[...])


def _tc_final(a2p, h0, x, touch):
    rb = 1000
    return pl.pallas_call(
        _tc_final_body,
        grid=(N // rb,),
        in_specs=[
            pl.BlockSpec((2, rb, EMB), lambda i: (0, i, 0)),
            pl.BlockSpec((rb, EMB), lambda i: (i, 0)),
            pl.BlockSpec((rb, EMB), lambda i: (i, 0)),
            pl.BlockSpec((NW, 32), lambda i: (0, 0)),
        ],
        out_specs=pl.BlockSpec((rb, EMB), lambda i: (i, 0)),
        out_shape=jax.ShapeDtypeStruct((N, EMB), F32),
    )(a2p, h0, x, touch)


def kernel(x, edge_index, edge_type, basis1, comp1, root1, bias1,
           basis2, comp2, root2, bias2):
    src = edge_index[0].astype(I32)
    dst = edge_index[1].astype(I32)
    typ = edge_type.astype(I32)
    pad = E_PAD - E
    src = jnp.pad(src, (0, pad))
    dst = jnp.pad(dst, (0, pad))
    typ = jnp.pad(typ, (0, pad))

    w1cat = jnp.einsum('rb,bio->iro', comp1, basis1).reshape(EMB, REL * HID)
    w2cat = jnp.einsum('rb,bio->iro', comp2, basis2).reshape(HID, REL * EMB)

    y1 = _tc_prep(x, w1cat)
    alphaf, alphac, touch = _sc_counts(src, dst, typ)
    apart = _sc_conv1(src, dst, typ, y1.reshape(N * REL // 2, 2 * HID),
                      alphaf, alphac)
    y2, h0 = _tc_mid(apart, x, root1, bias1.reshape(1, HID), w2cat, root2,
                     bias2.reshape(1, EMB))
    a2p = _sc_conv2(src, dst, typ, y2.reshape(2 * N * REL, EMB), alphaf)
    return _tc_final(a2p, h0[0], x, touch.reshape(NW, 32))


# speedup vs baseline: 6.2835x; 1.2807x over previous
import functools

import jax
import jax.numpy as jnp
from jax import lax
from jax.experimental import pallas as pl
from jax.experimental.pallas import tpu as pltpu
from jax.experimental.pallas import tpu_sc as plsc

N = 10000
REL = 40
EMB = 128
HID = 64
E = 160000
CHUNK = 5000
NSEG = N * REL
NC, NS = 2, 16
NW = NC * NS
E_PAD = 163840
CPT = E_PAD // NS
EPW = E_PAD // NW
CSTRIPE = NSEG // NS
ABIG = 640
ALAST = N - 15 * ABIG
F32 = jnp.float32
I32 = jnp.int32

_mesh = plsc.VectorSubcoreMesh(core_axis_name="c", subcore_axis_name="s")


def _z16():
    return jnp.zeros((16,), F32)


def _fill_zero_1d(ref, n):
    def body(i, _):
        ref[pl.ds(i * 16, 16)] = _z16()
        return 0
    lax.fori_loop(0, n // 16, body, 0)


def _fill_zero_2d(ref, rows, cols):
    def body(i, _):
        r = i // (cols // 16)
        q = i % (cols // 16)
        ref[r, pl.ds(q * 16, 16)] = _z16()
        return 0
    lax.fori_loop(0, rows * (cols // 16), body, 0)


def _zero_stripe_2d(zrows, tab, sid):
    @pl.when(sid < 15)
    def _():
        for j in range(5):
            pltpu.sync_copy(zrows, tab.at[pl.ds(sid * ABIG + j * 128, 128)])

    @pl.when(sid == 15)
    def _():
        for j in range(3):
            pltpu.sync_copy(zrows, tab.at[pl.ds(15 * ABIG + j * 128, 128)])
        pltpu.sync_copy(zrows.at[pl.ds(0, 16)],
                        tab.at[pl.ds(15 * ABIG + 384, 16)])


def _copy_stripe_out(tab, out, sid):
    @pl.when(sid < 15)
    def _():
        pltpu.sync_copy(tab.at[pl.ds(sid * ABIG, ABIG)],
                        out.at[pl.ds(sid * ABIG, ABIG)])

    @pl.when(sid == 15)
    def _():
        pltpu.sync_copy(tab.at[pl.ds(15 * ABIG, ALAST)],
                        out.at[pl.ds(15 * ABIG, ALAST)])


@functools.partial(
    pl.kernel,
    out_type=(
        jax.ShapeDtypeStruct((E_PAD,), F32),
        jax.ShapeDtypeStruct((E_PAD,), F32),
        jax.ShapeDtypeStruct((NW * 32,), F32),
    ),
    mesh=_mesh,
    scratch_types=(
        pltpu.VMEM_SHARED((NSEG,), F32),
        pltpu.VMEM((CPT,), I32),
        pltpu.VMEM((CPT,), I32),
        pltpu.VMEM((CPT,), I32),
        pltpu.VMEM((CPT,), F32),
        pltpu.VMEM((8000,), F32),
        pltpu.VMEM((2048,), I32),
        pltpu.VMEM((2048,), F32),
        pltpu.VMEM((2048,), F32),
        pltpu.VMEM((2048,), F32),
        pltpu.VMEM((32,), F32),
    ),
)
def _sc_counts(src_hbm, dst_hbm, typ_hbm,
               af_hbm, ac_hbm, touch_hbm,
               ctab, srcb, dstb, typb, wantall, zc,
               segc, valc, cfb, alb, touchb):
    cid = lax.axis_index("c")
    sid = lax.axis_index("s")
    iota16 = lax.iota(I32, 16)

    _fill_zero_1d(zc, 8000)
    for j in range(3):
        pltpu.sync_copy(zc.at[pl.ds(0, 8000)],
                        ctab.at[pl.ds(sid * CSTRIPE + j * 8000, 8000)])
    pltpu.sync_copy(zc.at[pl.ds(0, 1000)],
                    ctab.at[pl.ds(sid * CSTRIPE + 24000, 1000)])
    ebase = sid * CPT
    pltpu.sync_copy(src_hbm.at[pl.ds(ebase, CPT)], srcb)
    pltpu.sync_copy(dst_hbm.at[pl.ds(ebase, CPT)], dstb)
    pltpu.sync_copy(typ_hbm.at[pl.ds(ebase, CPT)], typb)
    plsc.subcore_barrier()

    def count_chunk(ch, touch):
        t0a, t1a = touch
        base = ch * 2048

        def body(i, carry):
            a0, a1 = carry
            off = base + i * 16
            s = srcb[pl.ds(off, 16)]
            d = dstb[pl.ds(off, 16)]
            t = typb[pl.ds(off, 16)]
            segc[pl.ds(i * 16, 16)] = d * REL + t
            valid = (ebase + off + iota16) < E
            crossb = (s >= CHUNK) != (d >= CHUNK)
            want = jnp.where(cid == 0, valid, valid & crossb)
            wf = jnp.where(want, 1.0, 0.0)
            valc[pl.ds(i * 16, 16)] = wf
            wantall[pl.ds(off, 16)] = wf
            a0 = a0 + jnp.where(((s < CHUNK) | (d < CHUNK)) & valid, 1.0, 0.0)
            a1 = a1 + jnp.where(((s >= CHUNK) | (d >= CHUNK)) & valid, 1.0, 0.0)
            return a0, a1

        t0a, t1a = lax.fori_loop(0, 128, body, (t0a, t1a))
        pltpu.sync_copy(valc, ctab.at[segc], add=True)
        return t0a, t1a

    t0a, t1a = lax.fori_loop(0, CPT // 2048, count_chunk, (_z16(), _z16()))
    zero_on_sc1 = jnp.where(cid == 0, 1.0, 0.0)
    touchb[pl.ds(0, 16)] = zero_on_sc1 * t0a
    touchb[pl.ds(16, 16)] = zero_on_sc1 * t1a
    pltpu.sync_copy(touchb, touch_hbm.at[pl.ds((sid * NC + cid) * 32, 32)])
    plsc.subcore_barrier()

    def alpha_chunk(ch, _):
        base = ch * 2048

        def mkseg(i, _c):
            off = base + i * 16
            d = dstb[pl.ds(off, 16)]
            t = typb[pl.ds(off, 16)]
            segc[pl.ds(i * 16, 16)] = d * REL + t
            return 0

        lax.fori_loop(0, 128, mkseg, 0)
        pltpu.sync_copy(ctab.at[segc], cfb)

        def mkalpha(i, _c):
            c = cfb[pl.ds(i * 16, 16)]
            w = wantall[pl.ds(base + i * 16, 16)]
            alb[pl.ds(i * 16, 16)] = w / jnp.maximum(c, 1.0)
            return 0

        lax.fori_loop(0, 128, mkalpha, 0)

        @pl.when(cid == 0)
        def _():
            pltpu.sync_copy(alb, af_hbm.at[pl.ds(ebase + base, 2048)])

        @pl.when(cid == 1)
        def _():
            pltpu.sync_copy(alb, ac_hbm.at[pl.ds(ebase + base, 2048)])
        return 0

    lax.fori_loop(0, CPT // 2048, alpha_chunk, 0)


CH1 = 128
NCH = EPW // CH1


@functools.partial(
    pl.kernel,
    out_type=jax.ShapeDtypeStruct((NC, N, 2 * HID), F32),
    mesh=_mesh,
    scratch_types=(
        pltpu.VMEM_SHARED((N, 2 * HID), F32),
        pltpu.VMEM((2, CH1), I32),
        pltpu.VMEM((2, CH1), I32),
        pltpu.VMEM((2, CH1), I32),
        pltpu.VMEM((2, CH1), F32),
        pltpu.VMEM((2, CH1), F32),
        pltpu.VMEM((2, CH1), I32),
        pltpu.VMEM((2, CH1), I32),
        pltpu.VMEM((2, CH1, 2 * HID), F32),
        pltpu.SemaphoreType.DMA,
        pltpu.SemaphoreType.DMA,
        pltpu.SemaphoreType.DMA,
        pltpu.SemaphoreType.DMA,
    ),
)
def _sc_conv1(src_hbm, dst_hbm, typ_hbm, y1_hbm, af_hbm, ac_hbm,
              apart_hbm,
              atab, srcb, dstb, typb, afb, acb, gidx, halfb, rows,
              seml0, seml1, semg0, semg1):
    cid = lax.axis_index("c")
    sid = lax.axis_index("s")
    seml = (seml0, seml1)
    semg = (semg0, semg1)

    _fill_zero_2d(rows.at[0], CH1, 2 * HID)
    _zero_stripe_2d(rows.at[0], atab, sid)
    ebase = (cid * NS + sid) * EPW
    plsc.subcore_barrier()

    def start_load(ch, b):
        base = ebase + ch * CH1
        pltpu.async_copy(src_hbm.at[pl.ds(base, CH1)], srcb.at[b], seml[b])
        pltpu.async_copy(dst_hbm.at[pl.ds(base, CH1)], dstb.at[b], seml[b])
        pltpu.async_copy(typ_hbm.at[pl.ds(base, CH1)], typb.at[b], seml[b])
        pltpu.async_copy(af_hbm.at[pl.ds(base, CH1)], afb.at[b], seml[b])
        pltpu.async_copy(ac_hbm.at[pl.ds(base, CH1)], acb.at[b], seml[b])

    def wait_load(b):
        base = ebase
        pltpu.make_async_copy(src_hbm.at[pl.ds(base, CH1)], srcb.at[b],
                              seml[b]).wait()
        pltpu.make_async_copy(dst_hbm.at[pl.ds(base, CH1)], dstb.at[b],
                              seml[b]).wait()
        pltpu.make_async_copy(typ_hbm.at[pl.ds(base, CH1)], typb.at[b],
                              seml[b]).wait()
        pltpu.make_async_copy(af_hbm.at[pl.ds(base, CH1)], afb.at[b],
                              seml[b]).wait()
        pltpu.make_async_copy(ac_hbm.at[pl.ds(base, CH1)], acb.at[b],
                              seml[b]).wait()

    def prep(b):
        def body(i, _c):
            s = srcb[b, pl.ds(i * 16, 16)]
            t = typb[b, pl.ds(i * 16, 16)]
            gidx[b, pl.ds(i * 16, 16)] = lax.shift_right_logical(t, 1) * N + s
            halfb[b, pl.ds(i * 16, 16)] = (t & 1) * HID
            return 0
        lax.fori_loop(0, CH1 // 16, body, 0)

    def start_gather(b):
        pltpu.async_copy(y1_hbm.at[gidx.at[b]], rows.at[b], semg[b])

    def wait_gather(b):
        pltpu.make_async_copy(y1_hbm.at[gidx.at[b]], rows.at[b],
                              semg[b]).wait()

    def scale_scatter(b):
        def body(grp, _c):
            av = afb[b, pl.ds(grp * 16, 16)]
            bv = acb[b, pl.ds(grp * 16, 16)]
            hv = halfb[b, pl.ds(grp * 16, 16)]
            for l in range(16):
                e = grp * 16 + l
                a = av[l]
                bb = bv[l]
                h = hv[l]
                for q in range(HID // 16):
                    v = rows[b, e, pl.ds(h + q * 16, 16)]
                    rows[b, e, pl.ds(q * 16, 16)] = v * a
                    rows[b, e, pl.ds(HID + q * 16, 16)] = v * bb
            return 0
        lax.fori_loop(0, CH1 // 16, body, 0)
        pltpu.sync_copy(rows.at[b], atab.at[dstb.at[b]], add=True)

    start_load(0, 0)

    def pipe(j, _):
        wait_load(0)
        prep(0)
        start_gather(0)

        @pl.when(j >= 1)
        def _():
            wait_gather(1)
            scale_scatter(1)
        start_load(2 * j + 1, 1)
        wait_load(1)
        prep(1)
        start_gather(1)
        wait_gather(0)
        scale_scatter(0)

        @pl.when(j + 1 < NCH // 2)
        def _():
            start_load(2 * j + 2, 0)
        return 0

    lax.fori_loop(0, NCH // 2, pipe, 0)
    wait_gather(1)
    scale_scatter(1)

    plsc.subcore_barrier()
    _copy_stripe_out(atab, apart_hbm.at[cid], sid)


@functools.partial(
    pl.kernel,
    out_type=jax.ShapeDtypeStruct((NC, N, EMB), F32),
    mesh=_mesh,
    scratch_types=(
        pltpu.VMEM_SHARED((N, EMB), F32),
        pltpu.VMEM((2, CH1), I32),
        pltpu.VMEM((2, CH1), I32),
        pltpu.VMEM((2, CH1), I32),
        pltpu.VMEM((2, CH1), F32),
        pltpu.VMEM((2, CH1), I32),
        pltpu.VMEM((2, CH1, EMB), F32),
        pltpu.SemaphoreType.DMA,
        pltpu.SemaphoreType.DMA,
        pltpu.SemaphoreType.DMA,
        pltpu.SemaphoreType.DMA,
    ),
)
def _sc_conv2(src_hbm, dst_hbm, typ_hbm, y2_hbm, af_hbm,
              a2_hbm,
              a2tab, srcb, dstb, typb, afb, gidx, rows,
              seml0, seml1, semg0, semg1):
    cid = lax.axis_index("c")
    sid = lax.axis_index("s")
    seml = (seml0, seml1)
    semg = (semg0, semg1)

    _fill_zero_2d(rows.at[0], CH1, EMB)
    _zero_stripe_2d(rows.at[0], a2tab, sid)
    ebase = (cid * NS + sid) * EPW
    plsc.subcore_barrier()

    def start_load(ch, b):
        base = ebase + ch * CH1
        pltpu.async_copy(src_hbm.at[pl.ds(base, CH1)], srcb.at[b], seml[b])
        pltpu.async_copy(dst_hbm.at[pl.ds(base, CH1)], dstb.at[b], seml[b])
        pltpu.async_copy(typ_hbm.at[pl.ds(base, CH1)], typb.at[b], seml[b])
        pltpu.async_copy(af_hbm.at[pl.ds(base, CH1)], afb.at[b], seml[b])

    def wait_load(b):
        pltpu.make_async_copy(src_hbm.at[pl.ds(ebase, CH1)], srcb.at[b],
                              seml[b]).wait()
        pltpu.make_async_copy(dst_hbm.at[pl.ds(ebase, CH1)], dstb.at[b],
                              seml[b]).wait()
        pltpu.make_async_copy(typ_hbm.at[pl.ds(ebase, CH1)], typb.at[b],
                              seml[b]).wait()
        pltpu.make_async_copy(af_hbm.at[pl.ds(ebase, CH1)], afb.at[b],
                              seml[b]).wait()

    def prep(b):
        def body(i, _c):
            s = srcb[b, pl.ds(i * 16, 16)]
            d = dstb[b, pl.ds(i * 16, 16)]
            t = typb[b, pl.ds(i * 16, 16)]
            crossb = (s >= CHUNK) != (d >= CHUNK)
            s2 = jnp.where(crossb, s + N, s)
            gidx[b, pl.ds(i * 16, 16)] = t * (2 * N) + s2
            return 0
        lax.fori_loop(0, CH1 // 16, body, 0)

    def start_gather(b):
        pltpu.async_copy(y2_hbm.at[gidx.at[b]], rows.at[b], semg[b])

    def wait_gather(b):
        pltpu.make_async_copy(y2_hbm.at[gidx.at[b]], rows.at[b],
                              semg[b]).wait()

    def scale_scatter(b):
        def body(grp, _c):
            av = afb[b, pl.ds(grp * 16, 16)]
            for l in range(16):
                e = grp * 16 + l
                a = av[l]
                for q in range(EMB // 16):
                    rows[b, e, pl.ds(q * 16, 16)] = (
                        rows[b, e, pl.ds(q * 16, 16)] * a)
            return 0
        lax.fori_loop(0, CH1 // 16, body, 0)
        pltpu.sync_copy(rows.at[b], a2tab.at[dstb.at[b]], add=True)

    start_load(0, 0)

    def pipe(j, _):
        wait_load(0)
        prep(0)
        start_gather(0)

        @pl.when(j >= 1)
        def _():
            wait_gather(1)
            scale_scatter(1)
        start_load(2 * j + 1, 1)
        wait_load(1)
        prep(1)
        start_gather(1)
        wait_gather(0)
        scale_scatter(0)

        @pl.when(j + 1 < NCH // 2)
        def _():
            start_load(2 * j + 2, 0)
        return 0

    lax.fori_loop(0, NCH // 2, pipe, 0)
    wait_gather(1)
    scale_scatter(1)

    plsc.subcore_barrier()
    _copy_stripe_out(a2tab, a2_hbm.at[cid], sid)


def _tc_prep_body(x_ref, w1_ref, y1_ref):
    y1_ref[0] = jnp.dot(x_ref[...], w1_ref[...], preferred_element_type=F32)


def _tc_prep(x, w1cat):
    rb = 1000
    return pl.pallas_call(
        _tc_prep_body,
        grid=(N // rb, REL // 2),
        in_specs=[
            pl.BlockSpec((rb, EMB), lambda i, p: (i, 0)),
            pl.BlockSpec((EMB, EMB), lambda i, p: (0, p)),
        ],
        out_specs=pl.BlockSpec((1, rb, EMB), lambda i, p: (p, i, 0)),
        out_shape=jax.ShapeDtypeStruct((REL // 2, N, EMB), F32),
    )(x, w1cat)


def _tc_mid_body(a_ref, x_ref, r1_ref, b1_ref, w2_ref, r2_ref, b2_ref,
                 y2_ref, h0_ref):
    k = pl.program_id(0)
    asum = a_ref[0] + a_ref[1]
    agg = jnp.where(k == 0, asum[:, :HID], asum[:, HID:])
    xr1 = jnp.dot(x_ref[...], r1_ref[...], preferred_element_type=F32)
    t = jnp.maximum(agg + xr1 + b1_ref[...], 0.0)
    y2 = jnp.dot(t, w2_ref[...], preferred_element_type=F32)
    for r in range(REL):
        y2_ref[r] = y2[:, r * EMB:(r + 1) * EMB]
    h0_ref[0] = jnp.dot(t, r2_ref[...], preferred_element_type=F32) + b2_ref[...]


def _tc_mid(apart, x, root1, bias1, w2cat, root2, bias2):
    rb = 400
    return pl.pallas_call(
        _tc_mid_body,
        grid=(2, N // rb),
        in_specs=[
            pl.BlockSpec((2, rb, 2 * HID), lambda k, i: (0, i, 0)),
            pl.BlockSpec((rb, EMB), lambda k, i: (i, 0)),
            pl.BlockSpec((EMB, HID), lambda k, i: (0, 0)),
            pl.BlockSpec((1, HID), lambda k, i: (0, 0)),
            pl.BlockSpec((HID, REL * EMB), lambda k, i: (0, 0)),
            pl.BlockSpec((HID, EMB), lambda k, i: (0, 0)),
            pl.BlockSpec((1, EMB), lambda k, i: (0, 0)),
        ],
        out_specs=[
            pl.BlockSpec((REL, rb, EMB),
                         lambda k, i: (0, k * (N // 400) + i, 0)),
            pl.BlockSpec((1, rb, EMB), lambda k, i: (k, i, 0)),
        ],
        out_shape=[
            jax.ShapeDtypeStruct((REL, 2 * N, EMB), F32),
            jax.ShapeDtypeStruct((2, N, EMB), F32),
        ],
    )(apart, x, root1, bias1, w2cat, root2, bias2)


def _tc_final_body(a2_ref, h0_ref, x_ref, touch_ref, h_ref):
    tch = touch_ref[...]
    t0 = jnp.sum(tch[:, :16]) > 0.0
    t1 = jnp.sum(tch[:, 16:]) > 0.0
    i = pl.program_id(0)
    tk = jnp.where(i < CHUNK // 1000, t0, t1)
    val = a2_ref[0] + a2_ref[1] + h0_ref[...]
    h_ref[...] = jnp.where(tk, val, x_ref[...])


def _tc_final(a2p, h0, x, touch):
    rb = 1000
    return pl.pallas_call(
        _tc_final_body,
        grid=(N // rb,),
        in_specs=[
            pl.BlockSpec((2, rb, EMB), lambda i: (0, i, 0)),
            pl.BlockSpec((rb, EMB), lambda i: (i, 0)),
            pl.BlockSpec((rb, EMB), lambda i: (i, 0)),
            pl.BlockSpec((NW, 32), lambda i: (0, 0)),
        ],
        out_specs=pl.BlockSpec((rb, EMB), lambda i: (i, 0)),
        out_shape=jax.ShapeDtypeStruct((N, EMB), F32),
    )(a2p, h0, x, touch)


def kernel(x, edge_index, edge_type, basis1, comp1, root1, bias1,
           basis2, comp2, root2, bias2):
    src = edge_index[0].astype(I32)
    dst = edge_index[1].astype(I32)
    typ = edge_type.astype(I32)
    pad = E_PAD - E
    src = jnp.pad(src, (0, pad))
    dst = jnp.pad(dst, (0, pad))
    typ = jnp.pad(typ, (0, pad))

    w1cat = jnp.einsum('rb,bio->iro', comp1, basis1).reshape(EMB, REL * HID)
    w2cat = jnp.einsum('rb,bio->iro', comp2, basis2).reshape(HID, REL * EMB)

    y1 = _tc_prep(x, w1cat)
    alphaf, alphac, touch = _sc_counts(src, dst, typ)
    apart = _sc_conv1(src, dst, typ, y1.reshape(N * REL // 2, 2 * HID),
                      alphaf, alphac)
    y2, h0 = _tc_mid(apart, x, root1, bias1.reshape(1, HID), w2cat, root2,
                     bias2.reshape(1, EMB))
    a2p = _sc_conv2(src, dst, typ, y2.reshape(2 * N * REL, EMB), alphaf)
    return _tc_final(a2p, h0[0], x, touch.reshape(NW, 32))
